# baseline (speedup 1.0000x reference)
import functools
import math

import jax
import jax.numpy as jnp
from jax import lax
from jax.experimental import pallas as pl
from jax.experimental.pallas import tpu as pltpu
from jax.experimental.pallas import tpu_sc as plsc

_B, _C1, _C, _N, _T = 8, 8, 32, 500, 24
_O = 10
_K = 5
_KP = 8
_INV_SQRT10 = 1.0 / math.sqrt(10.0)
_G = _B * _C1
_LANES = 16
_RB = (_N + _LANES - 1) // _LANES


def _tc_body(A_ref, mask_ref, W_ref, b_ref, a1_ref, a2_ref,
             acc_ref, idx_ref):
    N = _N
    Af = A_ref[...]
    rows = lax.broadcasted_iota(jnp.int32, (N, N), 0)
    cols = lax.broadcasted_iota(jnp.int32, (N, N), 1)
    colf = cols.astype(jnp.float32)
    colk = lax.broadcasted_iota(jnp.int32, (N, _KP), 1)
    big = jnp.float32(1e9)
    alpha_term = 0.001 * (
        jnp.where(cols > rows, a1_ref[...], 0.0)
        + jnp.where(cols <= rows, a2_ref[...], 0.0)
    )
    bias = b_ref[0]
    W = W_ref[...]

    acc = jnp.zeros((N, N), jnp.float32)
    for c1 in range(_C1):
        m = mask_ref[0, c1]
        P = lax.dot_general(m, W, (((1,), (1,)), ((), ())),
                            preferred_element_type=jnp.float32) + bias
        S = lax.dot_general(P, P, (((1,), (1,)), ((), ())),
                            preferred_element_type=jnp.float32)
        mw = jax.nn.sigmoid((S + alpha_term) * _INV_SQRT10)
        A4 = Af + 0.002 * mw
        acc = acc + A4

        a = A4
        idx_arr = jnp.zeros((N, _KP), jnp.int32)
        for j in range(_K):
            idxj = jnp.argmax(a, axis=1).reshape(N, 1)
            idx_arr = jnp.where(colk == j, idxj, idx_arr)
            if j + 1 < _K:
                a = jnp.where(cols == idxj, -jnp.inf, a)
        idx_ref[0, c1] = idx_arr

    acc_ref[0] = acc


def _tc_einsum_body(acc_ref, xt_ref, xo_ref):
    xo_ref[0] = jnp.dot(acc_ref[0], xt_ref[0],
                        preferred_element_type=jnp.float32)


def _sc_body(groups_per_w, num_cores, mask_hbm, idx_hbm, out_hbm,
             mask_v, idx_v, out_v):
    c = lax.axis_index("c")
    s = lax.axis_index("s")
    wid = s * num_cores + c
    lane = lax.iota(jnp.int32, _LANES)
    for gi in range(groups_per_w):
        g = wid * groups_per_w + gi
        pltpu.sync_copy(mask_hbm.at[g], mask_v)
        pltpu.sync_copy(idx_hbm.at[g], idx_v)

        def body(rb, carry):
            rvec = rb * _LANES + lane
            valid = rvec < _N
            rbase = rvec * _KP
            acc = [None] * _T
            for k in range(_K):
                jvec = plsc.load_gather(idx_v, [rbase + k], mask=valid)
                jb = jvec * _T
                for t in range(_T):
                    val = plsc.load_gather(mask_v, [jb + t], mask=valid)
                    acc[t] = val if k == 0 else jnp.maximum(acc[t], val)
            obase = rvec * _T
            for t in range(_T):
                plsc.store_scatter(out_v, [obase + t], acc[t], mask=valid)
            return carry

        lax.fori_loop(0, _RB, body, 0)
        pltpu.sync_copy(out_v, out_hbm.at[g])


@jax.jit
def kernel(x, A, mask, k, W_mlp, b_mlp, alpha1, alpha2):
    B, C, N, T = x.shape
    C1 = mask.shape[1]
    O = W_mlp.shape[0]
    x_t = jnp.transpose(x, (0, 2, 1, 3)).reshape(B, N, C * T)
    b2 = b_mlp.reshape(1, O)

    acc, topk_idx = pl.pallas_call(
        _tc_body,
        grid=(B,),
        in_specs=[
            pl.BlockSpec((N, N), lambda b: (0, 0)),
            pl.BlockSpec((1, C1, N, T), lambda b: (b, 0, 0, 0)),
            pl.BlockSpec((O, T), lambda b: (0, 0)),
            pl.BlockSpec((1, O), lambda b: (0, 0)),
            pl.BlockSpec((N, N), lambda b: (0, 0)),
            pl.BlockSpec((N, N), lambda b: (0, 0)),
        ],
        out_specs=[
            pl.BlockSpec((1, N, N), lambda b: (b, 0, 0)),
            pl.BlockSpec((1, C1, N, _KP), lambda b: (b, 0, 0, 0)),
        ],
        out_shape=[
            jax.ShapeDtypeStruct((B, N, N), jnp.float32),
            jax.ShapeDtypeStruct((B, C1, N, _KP), jnp.int32),
        ],
    )(A, mask, W_mlp, b2, alpha1, alpha2)

    x_out_t = pl.pallas_call(
        _tc_einsum_body,
        grid=(B,),
        in_specs=[
            pl.BlockSpec((1, N, N), lambda b: (b, 0, 0)),
            pl.BlockSpec((1, N, C * T), lambda b: (b, 0, 0)),
        ],
        out_specs=pl.BlockSpec((1, N, C * T), lambda b: (b, 0, 0)),
        out_shape=jax.ShapeDtypeStruct((B, N, C * T), jnp.float32),
    )(acc, x_t)

    x_out = x_out_t.reshape(B, N, C, T).transpose(0, 2, 1, 3)

    mesh = plsc.VectorSubcoreMesh(core_axis_name="c", subcore_axis_name="s")
    num_workers = mesh.num_cores * mesh.num_subcores
    groups_per_w = _G // num_workers
    mask2 = mask.reshape(_G, N * T)
    idx2 = topk_idx.reshape(_G, N * _KP)

    sc_fn = pl.kernel(
        functools.partial(_sc_body, groups_per_w, mesh.num_cores),
        out_type=jax.ShapeDtypeStruct((_G, N * T), jnp.float32),
        mesh=mesh,
        compiler_params=pltpu.CompilerParams(needs_layout_passes=False),
        scratch_types=[
            pltpu.VMEM((N * T,), jnp.float32),
            pltpu.VMEM((N * _KP,), jnp.int32),
            pltpu.VMEM((N * T,), jnp.float32),
        ],
    )
    mask_out = sc_fn(mask2, idx2).reshape(B, C1, N, T)
    return x_out, mask_out


# speedup vs baseline: 1.0686x; 1.0686x over previous
import functools
import math

import jax
import jax.numpy as jnp
from jax import lax
from jax.experimental import pallas as pl
from jax.experimental.pallas import tpu as pltpu
from jax.experimental.pallas import tpu_sc as plsc

_B, _C1, _C, _N, _T = 8, 8, 32, 500, 24
_O = 10
_K = 5
_KP = 8
_INV_SQRT10 = 1.0 / math.sqrt(10.0)
_G = _B * _C1
_LANES = 16
_RB = (_N + _LANES - 1) // _LANES


def _tc_body(A_ref, mask_ref, W_ref, b_ref, a1_ref, a2_ref,
             acc_ref, idx_ref):
    N = _N
    Af = A_ref[...]
    rows = lax.broadcasted_iota(jnp.int32, (N, N), 0)
    cols = lax.broadcasted_iota(jnp.int32, (N, N), 1)
    colf = cols.astype(jnp.float32)
    colk = lax.broadcasted_iota(jnp.int32, (N, _KP), 1)
    big = jnp.float32(1e9)
    alpha_term = 0.001 * (
        jnp.where(cols > rows, a1_ref[...], 0.0)
        + jnp.where(cols <= rows, a2_ref[...], 0.0)
    )
    bias = b_ref[0]
    W = W_ref[...]

    acc = jnp.zeros((N, N), jnp.float32)
    for c1 in range(_C1):
        m = mask_ref[0, c1]
        P = lax.dot_general(m, W, (((1,), (1,)), ((), ())),
                            preferred_element_type=jnp.float32) + bias
        S = lax.dot_general(P, P, (((1,), (1,)), ((), ())),
                            preferred_element_type=jnp.float32)
        mw = jax.nn.sigmoid((S + alpha_term) * _INV_SQRT10)
        A4 = Af + 0.002 * mw
        acc = acc + A4

        idx_arr = jnp.zeros((N, _KP), jnp.float32)
        picked = []
        for j in range(_K):
            am = A4
            for f in picked:
                am = jnp.where(colf == f, -jnp.inf, am)
            mx = jnp.max(am, axis=1, keepdims=True)
            first = jnp.min(jnp.where(am >= mx, colf, big), axis=1,
                            keepdims=True)
            idx_arr = jnp.where(colk == j, first, idx_arr)
            picked.append(first)
        idx_ref[0, c1] = idx_arr.astype(jnp.int32)

    acc_ref[0] = acc


def _tc_einsum_body(acc_ref, xt_ref, xo_ref):
    xo_ref[0] = jnp.dot(acc_ref[0], xt_ref[0],
                        preferred_element_type=jnp.float32)


def _sc_body(groups_per_w, num_cores, mask_hbm, idx_hbm, out_hbm,
             mask_v, idx_v, out_v):
    c = lax.axis_index("c")
    s = lax.axis_index("s")
    wid = s * num_cores + c
    lane = lax.iota(jnp.int32, _LANES)
    for gi in range(groups_per_w):
        g = wid * groups_per_w + gi
        pltpu.sync_copy(mask_hbm.at[g], mask_v)
        pltpu.sync_copy(idx_hbm.at[g], idx_v)

        def body(rb, carry):
            rvec = rb * _LANES + lane
            valid = rvec < _N
            rbase = rvec * _KP
            acc = [None] * _T
            for k in range(_K):
                jvec = plsc.load_gather(idx_v, [rbase + k], mask=valid)
                jb = jvec * _T
                for t in range(_T):
                    val = plsc.load_gather(mask_v, [jb + t], mask=valid)
                    acc[t] = val if k == 0 else jnp.maximum(acc[t], val)
            obase = rvec * _T
            for t in range(_T):
                plsc.store_scatter(out_v, [obase + t], acc[t], mask=valid)
            return carry

        lax.fori_loop(0, _RB, body, 0)
        pltpu.sync_copy(out_v, out_hbm.at[g])


@jax.jit
def kernel(x, A, mask, k, W_mlp, b_mlp, alpha1, alpha2):
    B, C, N, T = x.shape
    C1 = mask.shape[1]
    O = W_mlp.shape[0]
    x_t = jnp.transpose(x, (0, 2, 1, 3)).reshape(B, N, C * T)
    b2 = b_mlp.reshape(1, O)

    acc, topk_idx = pl.pallas_call(
        _tc_body,
        grid=(B,),
        in_specs=[
            pl.BlockSpec((N, N), lambda b: (0, 0)),
            pl.BlockSpec((1, C1, N, T), lambda b: (b, 0, 0, 0)),
            pl.BlockSpec((O, T), lambda b: (0, 0)),
            pl.BlockSpec((1, O), lambda b: (0, 0)),
            pl.BlockSpec((N, N), lambda b: (0, 0)),
            pl.BlockSpec((N, N), lambda b: (0, 0)),
        ],
        out_specs=[
            pl.BlockSpec((1, N, N), lambda b: (b, 0, 0)),
            pl.BlockSpec((1, C1, N, _KP), lambda b: (b, 0, 0, 0)),
        ],
        out_shape=[
            jax.ShapeDtypeStruct((B, N, N), jnp.float32),
            jax.ShapeDtypeStruct((B, C1, N, _KP), jnp.int32),
        ],
    )(A, mask, W_mlp, b2, alpha1, alpha2)

    x_out_t = pl.pallas_call(
        _tc_einsum_body,
        grid=(B,),
        in_specs=[
            pl.BlockSpec((1, N, N), lambda b: (b, 0, 0)),
            pl.BlockSpec((1, N, C * T), lambda b: (b, 0, 0)),
        ],
        out_specs=pl.BlockSpec((1, N, C * T), lambda b: (b, 0, 0)),
        out_shape=jax.ShapeDtypeStruct((B, N, C * T), jnp.float32),
    )(acc, x_t)

    x_out = x_out_t.reshape(B, N, C, T).transpose(0, 2, 1, 3)

    mesh = plsc.VectorSubcoreMesh(core_axis_name="c", subcore_axis_name="s")
    num_workers = mesh.num_cores * mesh.num_subcores
    groups_per_w = _G // num_workers
    mask2 = mask.reshape(_G, N * T)
    idx2 = topk_idx.reshape(_G, N * _KP)

    sc_fn = pl.kernel(
        functools.partial(_sc_body, groups_per_w, mesh.num_cores),
        out_type=jax.ShapeDtypeStruct((_G, N * T), jnp.float32),
        mesh=mesh,
        compiler_params=pltpu.CompilerParams(needs_layout_passes=False),
        scratch_types=[
            pltpu.VMEM((N * T,), jnp.float32),
            pltpu.VMEM((N * _KP,), jnp.int32),
            pltpu.VMEM((N * T,), jnp.float32),
        ],
    )
    mask_out = sc_fn(mask2, idx2).reshape(B, C1, N, T)
    return x_out, mask_out


# speedup vs baseline: 1.0691x; 1.0004x over previous
import functools
import math

import jax
import jax.numpy as jnp
from jax import lax
from jax.experimental import pallas as pl
from jax.experimental.pallas import tpu as pltpu
from jax.experimental.pallas import tpu_sc as plsc

_B, _C1, _C, _N, _T = 8, 8, 32, 500, 24
_O = 10
_K = 5
_KP = 8
_INV_SQRT10 = 1.0 / math.sqrt(10.0)
_G = _B * _C1
_LANES = 16
_RB = (_N + _LANES - 1) // _LANES


def _tc_body(A_ref, mask_ref, W_ref, b_ref, a1_ref, a2_ref,
             acc_ref, idx_ref):
    N = _N
    Af = A_ref[...]
    rows = lax.broadcasted_iota(jnp.int32, (N, N), 0)
    cols = lax.broadcasted_iota(jnp.int32, (N, N), 1)
    colf = cols.astype(jnp.float32)
    colk = lax.broadcasted_iota(jnp.int32, (N, _KP), 1)
    big = jnp.float32(1e9)
    alpha_term = 0.001 * (
        jnp.where(cols > rows, a1_ref[...], 0.0)
        + jnp.where(cols <= rows, a2_ref[...], 0.0)
    )
    bias = b_ref[0]
    W = W_ref[...]

    acc = jnp.zeros((N, N), jnp.float32)
    for c1 in range(_C1):
        m = mask_ref[0, c1]
        P = lax.dot_general(m, W, (((1,), (1,)), ((), ())),
                            preferred_element_type=jnp.float32) + bias
        S = lax.dot_general(P, P, (((1,), (1,)), ((), ())),
                            preferred_element_type=jnp.float32)
        mw = jax.nn.sigmoid((S + alpha_term) * _INV_SQRT10)
        A4 = Af + 0.002 * mw
        acc = acc + A4

        idx_arr = jnp.zeros((N, _KP), jnp.float32)
        picked = []
        for j in range(_K):
            am = A4
            for f in picked:
                am = jnp.where(colf == f, -jnp.inf, am)
            mx = jnp.max(am, axis=1, keepdims=True)
            first = jnp.min(jnp.where(am >= mx, colf, big), axis=1,
                            keepdims=True)
            idx_arr = jnp.where(colk == j, first, idx_arr)
            picked.append(first)
        idx_ref[0, c1] = idx_arr.astype(jnp.int32)

    acc_ref[0] = acc


def _tc_einsum_body(acc_ref, xt_ref, xo_ref):
    xo_ref[0] = jnp.dot(acc_ref[0], xt_ref[0],
                        preferred_element_type=jnp.float32)


def _sc_body(groups_per_w, num_cores, mask_hbm, idx_hbm, out_hbm,
             mask_v, idx_v, out_v):
    c = lax.axis_index("c")
    s = lax.axis_index("s")
    wid = s * num_cores + c
    lane = lax.iota(jnp.int32, _LANES)
    for gi in range(groups_per_w):
        g = wid * groups_per_w + gi
        pltpu.sync_copy(mask_hbm.at[g], mask_v)
        pltpu.sync_copy(idx_hbm.at[g], idx_v)

        def body(rb, carry):
            rvec = rb * _LANES + lane
            valid = rvec < _N
            rbase = rvec * _KP
            acc = [None] * _T
            for k in range(_K):
                jvec = plsc.load_gather(idx_v, [rbase + k], mask=valid)
                jb = jvec * _T
                for t in range(_T):
                    val = plsc.load_gather(mask_v, [jb + t], mask=valid)
                    acc[t] = val if k == 0 else jnp.maximum(acc[t], val)
            obase = rvec * _T
            for t in range(_T):
                plsc.store_scatter(out_v, [obase + t], acc[t], mask=valid)
            return carry

        lax.fori_loop(0, _RB, body, 0)
        pltpu.sync_copy(out_v, out_hbm.at[g])


@jax.jit
def kernel(x, A, mask, k, W_mlp, b_mlp, alpha1, alpha2):
    B, C, N, T = x.shape
    C1 = mask.shape[1]
    O = W_mlp.shape[0]
    x_t = jnp.transpose(x, (0, 2, 1, 3)).reshape(B, N, C * T)
    b2 = b_mlp.reshape(1, O)

    acc, topk_idx = pl.pallas_call(
        _tc_body,
        grid=(B,),
        in_specs=[
            pl.BlockSpec((N, N), lambda b: (0, 0)),
            pl.BlockSpec((1, C1, N, T), lambda b: (b, 0, 0, 0)),
            pl.BlockSpec((O, T), lambda b: (0, 0)),
            pl.BlockSpec((1, O), lambda b: (0, 0)),
            pl.BlockSpec((N, N), lambda b: (0, 0)),
            pl.BlockSpec((N, N), lambda b: (0, 0)),
        ],
        out_specs=[
            pl.BlockSpec((1, N, N), lambda b: (b, 0, 0)),
            pl.BlockSpec((1, C1, N, _KP), lambda b: (b, 0, 0, 0)),
        ],
        out_shape=[
            jax.ShapeDtypeStruct((B, N, N), jnp.float32),
            jax.ShapeDtypeStruct((B, C1, N, _KP), jnp.int32),
        ],
    )(A, mask, W_mlp, b2, alpha1, alpha2)

    mesh = plsc.VectorSubcoreMesh(core_axis_name="c", subcore_axis_name="s")
    num_workers = mesh.num_cores * mesh.num_subcores
    groups_per_w = _G // num_workers
    mask2 = mask.reshape(_G, N * T)
    idx2 = topk_idx.reshape(_G, N * _KP)

    sc_fn = pl.kernel(
        functools.partial(_sc_body, groups_per_w, mesh.num_cores),
        out_type=jax.ShapeDtypeStruct((_G, N * T), jnp.float32),
        mesh=mesh,
        compiler_params=pltpu.CompilerParams(needs_layout_passes=False),
        scratch_types=[
            pltpu.VMEM((N * T,), jnp.float32),
            pltpu.VMEM((N * _KP,), jnp.int32),
            pltpu.VMEM((N * T,), jnp.float32),
        ],
    )
    mask_out = sc_fn(mask2, idx2).reshape(B, C1, N, T)
    x_out_t = pl.pallas_call(
        _tc_einsum_body,
        grid=(B,),
        in_specs=[
            pl.BlockSpec((1, N, N), lambda b: (b, 0, 0)),
            pl.BlockSpec((1, N, C * T), lambda b: (b, 0, 0)),
        ],
        out_specs=pl.BlockSpec((1, N, C * T), lambda b: (b, 0, 0)),
        out_shape=jax.ShapeDtypeStruct((B, N, C * T), jnp.float32),
    )(acc, x_t)

    x_out = x_out_t.reshape(B, N, C, T).transpose(0, 2, 1, 3)

    return x_out, mask_out
